# Initial kernel scaffold; baseline (speedup 1.0000x reference)
import jax
import jax.numpy as jnp
from jax.experimental import pallas as pl


def kernel(hidden_states, gate_weight, gate_up_proj, down_proj):
    raise NotImplementedError("write your pallas kernel here")


# speedup vs baseline: 1.1165x; 1.1165x over previous
import functools

import jax
import jax.numpy as jnp
from jax import lax
from jax.experimental import pallas as pl
from jax.experimental.pallas import tpu as pltpu
from jax.experimental.pallas import tpu_sc as plsc

E = 8
TOPK = 2
BM = 256

NC = 2
NS = 16
NW = NC * NS


def _router_body(x_ref, gw_ref, pos_ref, wbc_ref, bexp_ref):
    T = x_ref.shape[0]
    nbpad = bexp_ref.shape[1]
    f32 = jnp.float32

    x = x_ref[...]
    gw = gw_ref[...]
    logits = lax.dot_general(x, gw, (((1,), (1,)), ((), ())),
                             preferred_element_type=f32)

    ut8 = (lax.broadcasted_iota(jnp.int32, (E, E), 0)
           <= lax.broadcasted_iota(jnp.int32, (E, E), 1)).astype(f32)
    l1 = jnp.max(logits, axis=1, keepdims=True)
    oh1 = (logits == l1).astype(f32)
    oh1 = oh1 * (lax.dot_general(oh1, ut8, (((1,), (0,)), ((), ())),
                                 preferred_element_type=f32) == 1.0)
    neg = jnp.float32(-1e30)
    masked = jnp.where(oh1 > 0, neg, logits)
    l2 = jnp.max(masked, axis=1, keepdims=True)
    oh2 = (masked == l2).astype(f32)
    oh2 = oh2 * (lax.dot_general(oh2, ut8, (((1,), (0,)), ((), ())),
                                 preferred_element_type=f32) == 1.0)

    w0 = jax.nn.sigmoid(l1 - l2)
    w1 = jax.nn.sigmoid(l2 - l1)
    wbc_ref[0] = jnp.broadcast_to(w0, (T, 16))
    wbc_ref[1] = jnp.broadcast_to(w1, (T, 16))

    lt = (lax.broadcasted_iota(jnp.int32, (T, T), 1)
          <= lax.broadcasted_iota(jnp.int32, (T, T), 0)).astype(f32)
    c1 = lax.dot_general(lt, oh1, (((1,), (0,)), ((), ())),
                         preferred_element_type=f32)
    c2 = lax.dot_general(lt, oh2, (((1,), (0,)), ((), ())),
                         preferred_element_type=f32)
    count1 = jnp.sum(oh1, axis=0, keepdims=True)
    count2 = jnp.sum(oh2, axis=0, keepdims=True)
    counts = count1 + count2

    bmf = jnp.float32(BM)
    padded = jnp.floor((counts + (bmf - 1.0)) / bmf) * bmf
    slt8 = (lax.broadcasted_iota(jnp.int32, (E, E), 0)
            < lax.broadcasted_iota(jnp.int32, (E, E), 1)).astype(f32)
    off = lax.dot_general(padded, slt8, (((1,), (0,)), ((), ())),
                          precision=lax.Precision.HIGHEST,
                          preferred_element_type=f32)

    pos0 = jnp.sum(oh1 * (off + c1 - 1.0), axis=1)
    pos1 = jnp.sum(oh2 * (off + count1 + c2 - 1.0), axis=1)
    pos_ref[0, :] = pos0.astype(jnp.int32)
    pos_ref[1, :] = pos1.astype(jnp.int32)

    i8 = (lax.broadcasted_iota(jnp.int32, (E, E), 0)
          == lax.broadcasted_iota(jnp.int32, (E, E), 1)).astype(f32)
    ones81 = jnp.ones((E, 1), f32)
    off_col = lax.dot_general(i8 * off, ones81, (((1,), (0,)), ((), ())),
                              precision=lax.Precision.HIGHEST,
                              preferred_element_type=f32)
    cnt_col = lax.dot_general(i8 * counts, ones81, (((1,), (0,)), ((), ())),
                              precision=lax.Precision.HIGHEST,
                              preferred_element_type=f32)
    bs_col = jnp.floor(off_col / bmf)
    nb_col = jnp.floor((cnt_col + (bmf - 1.0)) / bmf)
    bi = lax.broadcasted_iota(jnp.int32, (E, nbpad), 1)
    belongs = ((bi >= bs_col) & (bi < bs_col + nb_col)).astype(f32)
    erow = lax.broadcasted_iota(jnp.int32, (E, nbpad), 0)
    bexp = jnp.sum(erow * belongs, axis=0, keepdims=True)
    assigned = jnp.sum(belongs, axis=0, keepdims=True)
    bexp = jnp.where(assigned > 0, bexp, jnp.float32(E - 1))
    bexp_ref[...] = bexp.astype(jnp.int32)


def _dispatch_body(x_hbm, pos_hbm, xs_hbm, rows_v, idx0_v, idx1_v, sem0, sem1):
    T, H = x_hbm.shape
    ch = rows_v.shape[0]
    tpw = T // NW
    wid = lax.axis_index("s") * NC + lax.axis_index("c")
    base = wid * tpw
    for k in range(tpw // ch):
        t0 = base + k * ch
        pltpu.sync_copy(x_hbm.at[pl.ds(t0, ch), :], rows_v)
        pltpu.sync_copy(pos_hbm.at[pl.ds(t0, ch)], idx0_v)
        pltpu.sync_copy(pos_hbm.at[pl.ds(T + t0, ch)], idx1_v)
        d0 = pltpu.async_copy(rows_v, xs_hbm.at[idx0_v], sem0)
        d1 = pltpu.async_copy(rows_v, xs_hbm.at[idx1_v], sem1)
        d0.wait()
        d1.wait()


def _combine_body(ys_hbm, pos_hbm, wbc_hbm, out_hbm,
                  g0_v, g1_v, o_v, idx0_v, idx1_v, w0_v, w1_v, sem0, sem1):
    T, H = out_hbm.shape
    ct = g0_v.shape[0]
    tpw = T // NW
    wid = lax.axis_index("s") * NC + lax.axis_index("c")
    base = wid * tpw
    nch = H // 16
    for k in range(tpw // ct):
        t0 = base + k * ct
        pltpu.sync_copy(pos_hbm.at[pl.ds(t0, ct)], idx0_v)
        pltpu.sync_copy(pos_hbm.at[pl.ds(T + t0, ct)], idx1_v)
        pltpu.sync_copy(wbc_hbm.at[pl.ds(t0, ct), :], w0_v)
        pltpu.sync_copy(wbc_hbm.at[pl.ds(T + t0, ct), :], w1_v)
        d0 = pltpu.async_copy(ys_hbm.at[idx0_v], g0_v, sem0)
        d1 = pltpu.async_copy(ys_hbm.at[idx1_v], g1_v, sem1)
        d0.wait()
        d1.wait()
        for i in range(ct):
            wb0 = w0_v[i]
            wb1 = w1_v[i]

            def body(j, _):
                for u in range(4):
                    sl = pl.ds(j * 64 + u * 16, 16)
                    o_v[i, sl] = g0_v[i, sl] * wb0 + g1_v[i, sl] * wb1
                return 0

            lax.fori_loop(0, nch // 4, body, 0)
        pltpu.sync_copy(o_v, out_hbm.at[pl.ds(t0, ct), :])


def _mm_body(be_ref, xs_ref, gu_ref, dn_ref, ys_ref):
    I = dn_ref.shape[1]
    xb = xs_ref[...].astype(jnp.bfloat16)
    h1 = lax.dot_general(xb, gu_ref[0], (((1,), (0,)), ((), ())),
                         preferred_element_type=jnp.float32)
    g = h1[:, :I]
    u = h1[:, I:]
    act = (g * jax.nn.sigmoid(g) * u).astype(jnp.bfloat16)
    ys_ref[...] = lax.dot_general(act, dn_ref[0], (((1,), (0,)), ((), ())),
                                  preferred_element_type=jnp.float32)


def kernel(hidden_states, gate_weight, gate_up_proj, down_proj):
    b, s, h = hidden_states.shape
    e, _, i2 = gate_up_proj.shape
    i = i2 // 2
    T = b * s
    nb = (2 * T) // BM + e - 1
    pad = nb * BM
    nbpad = 128

    x = hidden_states.reshape(T, h)

    pos2, wbc3, bexp_row = pl.pallas_call(
        _router_body,
        out_shape=(
            jax.ShapeDtypeStruct((2, T), jnp.int32),
            jax.ShapeDtypeStruct((2, T, 16), jnp.float32),
            jax.ShapeDtypeStruct((1, nbpad), jnp.int32),
        ),
    )(x, gate_weight)
    pos = pos2.reshape(2 * T)
    wbc = wbc3.reshape(2 * T, 16)
    bexp = bexp_row.reshape(nbpad)[:nb]

    ch = 32
    mesh = plsc.VectorSubcoreMesh(core_axis_name="c", subcore_axis_name="s")
    xs = pl.kernel(
        _dispatch_body,
        out_type=jax.ShapeDtypeStruct((pad, h), jnp.float32),
        mesh=mesh,
        scratch_types=[
            pltpu.VMEM((ch, h), jnp.float32),
            pltpu.VMEM((ch,), jnp.int32),
            pltpu.VMEM((ch,), jnp.int32),
            pltpu.SemaphoreType.DMA,
            pltpu.SemaphoreType.DMA,
        ],
    )(x, pos)

    gu_bf = gate_up_proj.astype(jnp.bfloat16)
    dn_bf = down_proj.astype(jnp.bfloat16)

    grid_spec = pltpu.PrefetchScalarGridSpec(
        num_scalar_prefetch=1,
        grid=(nb,),
        in_specs=[
            pl.BlockSpec((BM, h), lambda b_, be: (b_, 0)),
            pl.BlockSpec((1, h, i2), lambda b_, be: (be[b_], 0, 0)),
            pl.BlockSpec((1, i, h), lambda b_, be: (be[b_], 0, 0)),
        ],
        out_specs=pl.BlockSpec((BM, h), lambda b_, be: (b_, 0)),
    )
    ys = pl.pallas_call(
        _mm_body,
        grid_spec=grid_spec,
        out_shape=jax.ShapeDtypeStruct((pad, h), jnp.float32),
    )(bexp, xs, gu_bf, dn_bf)

    ct = 16
    out = pl.kernel(
        _combine_body,
        out_type=jax.ShapeDtypeStruct((T, h), jnp.float32),
        mesh=mesh,
        scratch_types=[
            pltpu.VMEM((ct, h), jnp.float32),
            pltpu.VMEM((ct, h), jnp.float32),
            pltpu.VMEM((ct, h), jnp.float32),
            pltpu.VMEM((ct,), jnp.int32),
            pltpu.VMEM((ct,), jnp.int32),
            pltpu.VMEM((ct, 16), jnp.float32),
            pltpu.VMEM((ct, 16), jnp.float32),
            pltpu.SemaphoreType.DMA,
            pltpu.SemaphoreType.DMA,
        ],
    )(ys, pos, wbc)

    return out.reshape(b, s, h)


# speedup vs baseline: 1.4053x; 1.2587x over previous
import functools

import jax
import jax.numpy as jnp
from jax import lax
from jax.experimental import pallas as pl
from jax.experimental.pallas import tpu as pltpu
from jax.experimental.pallas import tpu_sc as plsc

E = 8
TOPK = 2
BM = 128

NC = 2
NS = 16
NW = NC * NS


def _router_body(x_ref, gw_ref, pos_ref, wbc_ref, bexp_ref):
    T = x_ref.shape[0]
    nbpad = bexp_ref.shape[1]
    f32 = jnp.float32

    x = x_ref[...]
    gw = gw_ref[...]
    logits = lax.dot_general(x, gw, (((1,), (1,)), ((), ())),
                             preferred_element_type=f32)

    ut8 = (lax.broadcasted_iota(jnp.int32, (E, E), 0)
           <= lax.broadcasted_iota(jnp.int32, (E, E), 1)).astype(f32)
    l1 = jnp.max(logits, axis=1, keepdims=True)
    oh1 = (logits == l1).astype(f32)
    oh1 = oh1 * (lax.dot_general(oh1, ut8, (((1,), (0,)), ((), ())),
                                 preferred_element_type=f32) == 1.0)
    neg = jnp.float32(-1e30)
    masked = jnp.where(oh1 > 0, neg, logits)
    l2 = jnp.max(masked, axis=1, keepdims=True)
    oh2 = (masked == l2).astype(f32)
    oh2 = oh2 * (lax.dot_general(oh2, ut8, (((1,), (0,)), ((), ())),
                                 preferred_element_type=f32) == 1.0)

    w0 = jax.nn.sigmoid(l1 - l2)
    w1 = jax.nn.sigmoid(l2 - l1)
    wbc_ref[0] = jnp.broadcast_to(w0, (T, 16))
    wbc_ref[1] = jnp.broadcast_to(w1, (T, 16))

    lt = (lax.broadcasted_iota(jnp.int32, (T, T), 1)
          <= lax.broadcasted_iota(jnp.int32, (T, T), 0)).astype(f32)
    c1 = lax.dot_general(lt, oh1, (((1,), (0,)), ((), ())),
                         preferred_element_type=f32)
    c2 = lax.dot_general(lt, oh2, (((1,), (0,)), ((), ())),
                         preferred_element_type=f32)
    count1 = jnp.sum(oh1, axis=0, keepdims=True)
    count2 = jnp.sum(oh2, axis=0, keepdims=True)
    counts = count1 + count2

    bmf = jnp.float32(BM)
    padded = jnp.floor((counts + (bmf - 1.0)) / bmf) * bmf
    slt8 = (lax.broadcasted_iota(jnp.int32, (E, E), 0)
            < lax.broadcasted_iota(jnp.int32, (E, E), 1)).astype(f32)
    off = lax.dot_general(padded, slt8, (((1,), (0,)), ((), ())),
                          precision=lax.Precision.HIGHEST,
                          preferred_element_type=f32)

    pos0 = jnp.sum(oh1 * (off + c1 - 1.0), axis=1)
    pos1 = jnp.sum(oh2 * (off + count1 + c2 - 1.0), axis=1)
    pos_ref[0, :] = pos0.astype(jnp.int32)
    pos_ref[1, :] = pos1.astype(jnp.int32)

    i8 = (lax.broadcasted_iota(jnp.int32, (E, E), 0)
          == lax.broadcasted_iota(jnp.int32, (E, E), 1)).astype(f32)
    ones81 = jnp.ones((E, 1), f32)
    off_col = lax.dot_general(i8 * off, ones81, (((1,), (0,)), ((), ())),
                              precision=lax.Precision.HIGHEST,
                              preferred_element_type=f32)
    cnt_col = lax.dot_general(i8 * counts, ones81, (((1,), (0,)), ((), ())),
                              precision=lax.Precision.HIGHEST,
                              preferred_element_type=f32)
    bs_col = jnp.floor(off_col / bmf)
    nb_col = jnp.floor((cnt_col + (bmf - 1.0)) / bmf)
    bi = lax.broadcasted_iota(jnp.int32, (E, nbpad), 1)
    belongs = ((bi >= bs_col) & (bi < bs_col + nb_col)).astype(f32)
    erow = lax.broadcasted_iota(jnp.int32, (E, nbpad), 0)
    bexp = jnp.sum(erow * belongs, axis=0, keepdims=True)
    assigned = jnp.sum(belongs, axis=0, keepdims=True)
    bexp = jnp.where(assigned > 0, bexp, jnp.float32(E - 1))
    bexp_ref[...] = bexp.astype(jnp.int32)


def _dispatch_body(x_hbm, pos_hbm, xs_hbm, rows0_v, rows1_v, idx_v,
                   semL0, semL1, semS):
    T, H = x_hbm.shape
    ch = rows0_v.shape[0]
    tpw = T // NW
    wid = lax.axis_index("s") * NC + lax.axis_index("c")
    base = wid * tpw
    nk = tpw // ch
    rows = (rows0_v, rows1_v)
    semsl = (semL0, semL1)
    pltpu.sync_copy(pos_hbm.at[pl.ds(base, tpw)], idx_v.at[0])
    pltpu.sync_copy(pos_hbm.at[pl.ds(T + base, tpw)], idx_v.at[1])
    loads = [None] * nk
    loads[0] = pltpu.async_copy(x_hbm.at[pl.ds(base, ch), :], rows[0], semsl[0])
    for k in range(nk):
        cur = k % 2
        if k + 1 < nk:
            loads[k + 1] = pltpu.async_copy(
                x_hbm.at[pl.ds(base + (k + 1) * ch, ch), :],
                rows[(k + 1) % 2], semsl[(k + 1) % 2])
        loads[k].wait()
        i0 = idx_v[0, pl.ds(k * ch, ch)]
        i1 = idx_v[1, pl.ds(k * ch, ch)]
        d0 = pltpu.async_copy(rows[cur], xs_hbm.at[i0], semS)
        d1 = pltpu.async_copy(rows[cur], xs_hbm.at[i1], semS)
        d0.wait()
        d1.wait()


def _combine_body(ys_hbm, pos_hbm, wbc_hbm, out_hbm,
                  g0a_v, g0b_v, g1a_v, g1b_v, oa_v, ob_v,
                  i0a_v, i0b_v, i1a_v, i1b_v, wb_v,
                  semGa, semGb, semOa, semOb):
    T, H = out_hbm.shape
    ct = g0a_v.shape[0]
    tpw = T // NW
    wid = lax.axis_index("s") * NC + lax.axis_index("c")
    base = wid * tpw
    nk = tpw // ct
    g0 = (g0a_v, g0b_v)
    g1 = (g1a_v, g1b_v)
    ov = (oa_v, ob_v)
    pltpu.sync_copy(wbc_hbm.at[pl.ds(base, tpw), :], wb_v.at[0])
    pltpu.sync_copy(wbc_hbm.at[pl.ds(T + base, tpw), :], wb_v.at[1])

    semg = (semGa, semGb)
    semo = (semOa, semOb)
    i0r = (i0a_v, i0b_v)
    i1r = (i1a_v, i1b_v)

    def gathers(k):
        s = k % 2
        pltpu.sync_copy(pos_hbm.at[pl.ds(base + k * ct, ct)], i0r[s])
        pltpu.sync_copy(pos_hbm.at[pl.ds(T + base + k * ct, ct)], i1r[s])
        return (pltpu.async_copy(ys_hbm.at[i0r[s]], g0[s], semg[s]),
                pltpu.async_copy(ys_hbm.at[i1r[s]], g1[s], semg[s]))

    pend = gathers(0)
    stores = [None] * nk
    for k in range(nk):
        s = k % 2
        nxt = None
        if k + 1 < nk:
            nxt = gathers(k + 1)
        pend[0].wait()
        pend[1].wait()
        pend = nxt
        if k >= 2:
            stores[k - 2].wait()
        for i in range(ct):
            wb0 = wb_v[0, k * ct + i]
            wb1 = wb_v[1, k * ct + i]

            def body(j, _):
                for u in range(4):
                    sl = pl.ds(j * 64 + u * 16, 16)
                    ov[s][i, sl] = g0[s][i, sl] * wb0 + g1[s][i, sl] * wb1
                return 0

            lax.fori_loop(0, H // 64, body, 0)
        stores[k] = pltpu.async_copy(
            ov[s], out_hbm.at[pl.ds(base + k * ct, ct), :], semo[s])
    stores[nk - 2].wait()
    stores[nk - 1].wait()


def _mm_body(be_ref, xs_ref, gu_ref, dn_ref, ys_ref):
    I = dn_ref.shape[1]
    h1 = lax.dot_general(xs_ref[...], gu_ref[0], (((1,), (0,)), ((), ())),
                         preferred_element_type=jnp.float32)
    g = h1[:, :I]
    u = h1[:, I:]
    act = g * jax.nn.sigmoid(g) * u
    ys_ref[...] = lax.dot_general(act, dn_ref[0], (((1,), (0,)), ((), ())),
                                  preferred_element_type=jnp.float32)


def kernel(hidden_states, gate_weight, gate_up_proj, down_proj):
    b, s, h = hidden_states.shape
    e, _, i2 = gate_up_proj.shape
    i = i2 // 2
    T = b * s
    nb = (2 * T) // BM + e - 1
    pad = nb * BM
    nbpad = 128

    x = hidden_states.reshape(T, h)

    pos2, wbc3, bexp_row = pl.pallas_call(
        _router_body,
        out_shape=(
            jax.ShapeDtypeStruct((2, T), jnp.int32),
            jax.ShapeDtypeStruct((2, T, 16), jnp.float32),
            jax.ShapeDtypeStruct((1, nbpad), jnp.int32),
        ),
    )(x, gate_weight)
    pos = pos2.reshape(2 * T)
    wbc = wbc3.reshape(2 * T, 16)
    bexp = bexp_row.reshape(nbpad)[:nb]

    ch = 16
    tpw = T // NW
    mesh = plsc.VectorSubcoreMesh(core_axis_name="c", subcore_axis_name="s")
    xs = pl.kernel(
        _dispatch_body,
        out_type=jax.ShapeDtypeStruct((pad, h), jnp.float32),
        mesh=mesh,
        scratch_types=[
            pltpu.VMEM((ch, h), jnp.float32),
            pltpu.VMEM((ch, h), jnp.float32),
            pltpu.VMEM((2, tpw), jnp.int32),
            pltpu.SemaphoreType.DMA,
            pltpu.SemaphoreType.DMA,
            pltpu.SemaphoreType.DMA,
        ],
    )(x, pos)

    grid_spec = pltpu.PrefetchScalarGridSpec(
        num_scalar_prefetch=1,
        grid=(nb,),
        in_specs=[
            pl.BlockSpec((BM, h), lambda b_, be: (b_, 0)),
            pl.BlockSpec((1, h, i2), lambda b_, be: (be[b_], 0, 0)),
            pl.BlockSpec((1, i, h), lambda b_, be: (be[b_], 0, 0)),
        ],
        out_specs=pl.BlockSpec((BM, h), lambda b_, be: (b_, 0)),
    )
    ys = pl.pallas_call(
        _mm_body,
        grid_spec=grid_spec,
        out_shape=jax.ShapeDtypeStruct((pad, h), jnp.float32),
    )(bexp, xs, gate_up_proj, down_proj)

    ct = 8
    out = pl.kernel(
        _combine_body,
        out_type=jax.ShapeDtypeStruct((T, h), jnp.float32),
        mesh=mesh,
        scratch_types=[
            pltpu.VMEM((ct, h), jnp.float32),
            pltpu.VMEM((ct, h), jnp.float32),
            pltpu.VMEM((ct, h), jnp.float32),
            pltpu.VMEM((ct, h), jnp.float32),
            pltpu.VMEM((ct, h), jnp.float32),
            pltpu.VMEM((ct, h), jnp.float32),
            pltpu.VMEM((ct,), jnp.int32),
            pltpu.VMEM((ct,), jnp.int32),
            pltpu.VMEM((ct,), jnp.int32),
            pltpu.VMEM((ct,), jnp.int32),
            pltpu.VMEM((2, tpw, 16), jnp.float32),
            pltpu.SemaphoreType.DMA,
            pltpu.SemaphoreType.DMA,
            pltpu.SemaphoreType.DMA,
            pltpu.SemaphoreType.DMA,
        ],
    )(ys, pos, wbc)

    return out.reshape(b, s, h)
